# baseline (speedup 1.0000x reference)
import functools

import jax
import jax.numpy as jnp
from jax import lax
from jax.experimental import pallas as pl
from jax.experimental.pallas import tpu as pltpu
from jax.experimental.pallas import tpu_sc as plsc

_NUM_EMB = 1000000
_EMB_DIM = 64
_B = 4096 * 50
_W = 131072
_NWIN = 8

_SCALE_COLS = 8192


def _scale_body(w_ref, out_ref, acc_ref):
    i = pl.program_id(0)

    @pl.when(i == 0)
    def _():
        acc_ref[0] = 0.0

    col = i * _SCALE_COLS + lax.broadcasted_iota(
        jnp.int32, (_EMB_DIM, _SCALE_COLS), 1
    )
    a = jnp.where(col < _NUM_EMB, jnp.abs(w_ref[...]), 0.0)
    acc_ref[0] = jnp.maximum(acc_ref[0], jnp.max(a))

    @pl.when(i == pl.num_programs(0) - 1)
    def _():
        out_ref[0] = jnp.maximum(acc_ref[0], 1e-8) / 127.0


_scale_call = pl.pallas_call(
    _scale_body,
    grid=(pl.cdiv(_NUM_EMB, _SCALE_COLS),),
    in_specs=[pl.BlockSpec((_EMB_DIM, _SCALE_COLS), lambda i: (0, i))],
    out_specs=pl.BlockSpec(memory_space=pltpu.SMEM),
    out_shape=jax.ShapeDtypeStruct((1,), jnp.float32),
    scratch_shapes=[pltpu.SMEM((1,), jnp.float32)],
)

_QT_COLS = 1024
_K = _EMB_DIM * _NWIN


def _quant_body(scale_ref, *refs):
    w_refs = refs[:_NWIN]
    out_ref, sa_ref, sb_ref = refs[_NWIN], refs[_NWIN + 1], refs[_NWIN + 2]
    i = pl.program_id(0)

    @pl.when(i == 0)
    def _():
        r = lax.broadcasted_iota(jnp.int32, (_K, 128), 0)
        n = lax.broadcasted_iota(jnp.int32, (_K, 128), 1)
        same = (r >> 6) == (n >> 4)
        l, j4 = r & 63, (n & 15) * 4
        sa_ref[...] = jnp.where(same & (l == j4), 1.0, 0.0) + jnp.where(
            same & (l == j4 + 1), 256.0, 0.0
        )
        sb_ref[...] = jnp.where(same & (l == j4 + 2), 1.0, 0.0) + jnp.where(
            same & (l == j4 + 3), 256.0, 0.0
        )

    inv = 1.0 / scale_ref[0]
    q = jnp.concatenate([r[...] for r in w_refs], axis=0)
    q = jnp.clip(jnp.round(q * inv), -127.0, 126.0) + 128.0
    wa = lax.dot_general(q, sa_ref[...], (((0,), (0,)), ((), ())))
    wb = lax.dot_general(q, sb_ref[...], (((0,), (0,)), ((), ())))
    w = (wa.astype(jnp.int32) & 0xFFFF) | (wb.astype(jnp.int32) << 16)
    out_ref[...] = w ^ jnp.int32(-2139062144)


_quant_call = pl.pallas_call(
    _quant_body,
    grid=(_W // _QT_COLS,),
    in_specs=[pl.BlockSpec(memory_space=pltpu.SMEM)]
    + [
        pl.BlockSpec(
            (_EMB_DIM, _QT_COLS),
            functools.partial(
                lambda s, i: (0, jnp.minimum(s * (_W // _QT_COLS) + i, 976)), s
            ),
        )
        for s in range(_NWIN)
    ],
    out_specs=pl.BlockSpec((_QT_COLS, 128), lambda i: (i, 0)),
    out_shape=jax.ShapeDtypeStruct((_W, 128), jnp.int32),
    scratch_shapes=[
        pltpu.VMEM((_K, 128), jnp.float32),
        pltpu.VMEM((_K, 128), jnp.float32),
    ],
)

_NW = 32
_B_PER_W = _B // _NW
_CHUNK = 640
_NCHUNK = _B_PER_W // _CHUNK


def _gather_body(table_hbm, idx_hbm, out_hbm, idx_v, p_v, h_v, rows_v, out_v, sem):
    wid = lax.axis_index("s") * 2 + lax.axis_index("c")
    base = wid * _B_PER_W
    lanes = lax.iota(jnp.int32, 16)

    def chunk(c, carry):
        off = pl.multiple_of(base + c * _CHUNK, 128)
        pltpu.sync_copy(idx_hbm.at[pl.ds(off, _CHUNK)], idx_v)

        def prep(g, carry2):
            s = pl.multiple_of(g * 16, 16)
            iv = idx_v[pl.ds(s, 16)]
            p_v[pl.ds(s, 16)] = iv & (_W - 1)
            h_v[pl.ds(s, 16)] = lax.shift_right_logical(iv, 17) << 4
            return carry2

        lax.fori_loop(0, _CHUNK // 16, prep, 0)
        pltpu.async_copy(table_hbm.at[p_v], rows_v, sem).wait()

        def select(g, carry2):
            s = pl.multiple_of(g * 16, 16)
            r_vec = s + lanes
            h16 = h_v[pl.ds(s, 16)]
            o_row = lax.shift_right_logical(r_vec, 3)
            o_col = (r_vec & 7) << 4
            for j in range(16):
                v = plsc.load_gather(rows_v, [r_vec, h16 + j])
                plsc.store_scatter(out_v, [o_row, o_col + j], v)
            return carry2

        lax.fori_loop(0, _CHUNK // 16, select, 0)
        oof = pl.multiple_of(lax.shift_right_logical(base + c * _CHUNK, 3), 16)
        pltpu.sync_copy(out_v, out_hbm.at[pl.ds(oof, _CHUNK // 8)])
        return carry

    lax.fori_loop(0, _NCHUNK, chunk, 0)


_gather_call = functools.partial(
    pl.kernel,
    mesh=plsc.VectorSubcoreMesh(core_axis_name="c", subcore_axis_name="s"),
    compiler_params=pltpu.CompilerParams(
        use_tc_tiling_on_sc=False, needs_layout_passes=False
    ),
    out_type=jax.ShapeDtypeStruct((_B // 8, 128), jnp.int32),
    scratch_types=[
        pltpu.VMEM((_CHUNK,), jnp.int32),
        pltpu.VMEM((_CHUNK,), jnp.int32),
        pltpu.VMEM((_CHUNK,), jnp.int32),
        pltpu.VMEM((_CHUNK, 128), jnp.int32),
        pltpu.VMEM((_CHUNK // 8, 128), jnp.int32),
        pltpu.SemaphoreType.DMA,
    ],
)(_gather_body)


def kernel(x, weight):
    wt = weight.T
    scale = _scale_call(wt)
    packed = _quant_call(scale, *([wt] * _NWIN))
    rows = _gather_call(packed, x.reshape(-1))
    emb = lax.bitcast_convert_type(rows, jnp.int8)
    return emb.reshape(x.shape[0], x.shape[1], _EMB_DIM), scale


# speedup vs baseline: 3.5398x; 3.5398x over previous
import functools

import jax
import jax.numpy as jnp
from jax import lax
from jax.experimental import pallas as pl
from jax.experimental.pallas import tpu as pltpu
from jax.experimental.pallas import tpu_sc as plsc

_NUM_EMB = 1000000
_EMB_DIM = 64
_B = 4096 * 50
_W = 131072
_NWIN = 8

_SCALE_COLS = 8192


def _scale_body(w_ref, out_ref, acc_ref):
    i = pl.program_id(0)

    @pl.when(i == 0)
    def _():
        acc_ref[0] = 0.0

    col = i * _SCALE_COLS + lax.broadcasted_iota(
        jnp.int32, (_EMB_DIM, _SCALE_COLS), 1
    )
    a = jnp.where(col < _NUM_EMB, jnp.abs(w_ref[...]), 0.0)
    acc_ref[0] = jnp.maximum(acc_ref[0], jnp.max(a))

    @pl.when(i == pl.num_programs(0) - 1)
    def _():
        out_ref[0] = jnp.maximum(acc_ref[0], 1e-8) / 127.0


_scale_call = pl.pallas_call(
    _scale_body,
    grid=(pl.cdiv(_NUM_EMB, _SCALE_COLS),),
    in_specs=[pl.BlockSpec((_EMB_DIM, _SCALE_COLS), lambda i: (0, i))],
    out_specs=pl.BlockSpec(memory_space=pltpu.SMEM),
    out_shape=jax.ShapeDtypeStruct((1,), jnp.float32),
    scratch_shapes=[pltpu.SMEM((1,), jnp.float32)],
)

_QT_COLS = 1024
_K = _EMB_DIM * _NWIN


def _quant_body(scale_ref, *refs):
    w_refs = refs[:_NWIN]
    out_ref, sa_ref, sb_ref = refs[_NWIN], refs[_NWIN + 1], refs[_NWIN + 2]
    i = pl.program_id(0)

    @pl.when(i == 0)
    def _():
        r = lax.broadcasted_iota(jnp.int32, (_K, 128), 0)
        n = lax.broadcasted_iota(jnp.int32, (_K, 128), 1)
        same = (r >> 6) == (n >> 4)
        l, j4 = r & 63, (n & 15) * 4
        sa_ref[...] = jnp.where(same & (l == j4), 1.0, 0.0) + jnp.where(
            same & (l == j4 + 1), 256.0, 0.0
        )
        sb_ref[...] = jnp.where(same & (l == j4 + 2), 1.0, 0.0) + jnp.where(
            same & (l == j4 + 3), 256.0, 0.0
        )

    inv = 1.0 / scale_ref[0]
    q = jnp.concatenate([r[...] for r in w_refs], axis=0)
    q = jnp.clip(jnp.round(q * inv), -127.0, 126.0) + 128.0
    wa = lax.dot_general(q, sa_ref[...], (((0,), (0,)), ((), ())))
    wb = lax.dot_general(q, sb_ref[...], (((0,), (0,)), ((), ())))
    w = (wa.astype(jnp.int32) & 0xFFFF) | (wb.astype(jnp.int32) << 16)
    out_ref[...] = w ^ jnp.int32(-2139062144)


_quant_call = pl.pallas_call(
    _quant_body,
    grid=(_W // _QT_COLS,),
    in_specs=[pl.BlockSpec(memory_space=pltpu.SMEM)]
    + [
        pl.BlockSpec(
            (_EMB_DIM, _QT_COLS),
            functools.partial(
                lambda s, i: (0, jnp.minimum(s * (_W // _QT_COLS) + i, 976)), s
            ),
        )
        for s in range(_NWIN)
    ],
    out_specs=pl.BlockSpec((_QT_COLS, 128), lambda i: (i, 0)),
    out_shape=jax.ShapeDtypeStruct((_W, 128), jnp.int32),
    scratch_shapes=[
        pltpu.VMEM((_K, 128), jnp.float32),
        pltpu.VMEM((_K, 128), jnp.float32),
    ],
)

_NW = 32
_B_PER_W = _B // _NW
_CHUNK = 640
_NCHUNK = _B_PER_W // _CHUNK


_HIST = 50
_CHUNK_B = 16
_LK = _HIST * _CHUNK_B
_SUB = 400


def _gather_body(table_hbm, idx_hbm, out_hbm, idx_v, p_v, h_v, rows_v, out_v, sem):
    wid = lax.axis_index("s") * 2 + lax.axis_index("c")
    base = wid * _B_PER_W
    lanes = lax.iota(jnp.int32, 16)

    def chunk(c, carry):
        off = pl.multiple_of(base + c * _LK, 32)
        pltpu.sync_copy(idx_hbm.at[pl.ds(off, _LK)], idx_v)

        def prep(g, carry2):
            s = pl.multiple_of(g * 16, 16)
            iv = idx_v[pl.ds(s, 16)]
            p_v[pl.ds(s, 16)] = iv & (_W - 1)
            h_v[pl.ds(s, 16)] = lax.shift_right_logical(iv, 17) << 4
            return carry2

        lax.fori_loop(0, _LK // 16, prep, 0)

        for sub in range(_LK // _SUB):
            pltpu.async_copy(
                table_hbm.at[p_v.at[pl.ds(sub * _SUB, _SUB)]], rows_v, sem
            ).wait()

            def select(g, carry2):
                s = pl.multiple_of(g * 16, 16)
                r_vec = s + lanes
                k_vec = sub * _SUB + r_vec
                h16 = h_v[pl.ds(sub * _SUB + s, 16)]
                b_loc = lax.shift_right_logical(k_vec * 1311, 16)
                hh = k_vec - b_loc * _HIST
                o_base = hh << 4
                for j in range(16):
                    v = plsc.load_gather(rows_v, [r_vec, h16 + j])
                    plsc.store_scatter(out_v, [o_base + j, b_loc], v)
                return carry2

            lax.fori_loop(0, _SUB // 16, select, 0)

        col0 = pl.multiple_of(wid * 128 + c * _CHUNK_B, 16)
        pltpu.sync_copy(out_v, out_hbm.at[:, pl.ds(col0, _CHUNK_B)])
        return carry

    lax.fori_loop(0, _B_PER_W // _LK, chunk, 0)


_gather_call = functools.partial(
    pl.kernel,
    mesh=plsc.VectorSubcoreMesh(core_axis_name="c", subcore_axis_name="s"),
    compiler_params=pltpu.CompilerParams(
        use_tc_tiling_on_sc=False, needs_layout_passes=False
    ),
    out_type=jax.ShapeDtypeStruct((_HIST * 16, 4096), jnp.int32),
    scratch_types=[
        pltpu.VMEM((_LK,), jnp.int32),
        pltpu.VMEM((_LK,), jnp.int32),
        pltpu.VMEM((_LK,), jnp.int32),
        pltpu.VMEM((_SUB, 128), jnp.int32),
        pltpu.VMEM((_HIST * 16, _CHUNK_B), jnp.int32),
        pltpu.SemaphoreType.DMA,
    ],
)(_gather_body)


def kernel(x, weight):
    wt = weight.T
    scale = _scale_call(wt)
    packed = _quant_call(scale, *([wt] * _NWIN))
    rows = _gather_call(packed, x.reshape(-1))
    emb8 = lax.bitcast_convert_type(rows, jnp.int8)
    emb = (
        emb8.reshape(_HIST, 16, 4096, 4)
        .transpose(2, 0, 1, 3)
        .reshape(4096, _HIST, _EMB_DIM)
    )
    return emb, scale


# speedup vs baseline: 3.8996x; 1.1016x over previous
import functools

import jax
import jax.numpy as jnp
from jax import lax
from jax.experimental import pallas as pl
from jax.experimental.pallas import tpu as pltpu
from jax.experimental.pallas import tpu_sc as plsc

_NUM_EMB = 1000000
_EMB_DIM = 64
_B = 4096 * 50
_W = 131072
_NWIN = 8

_SCALE_COLS = 16384


def _scale_body(w_ref, out_ref, acc_ref):
    i = pl.program_id(0)
    nlast = pl.num_programs(0) - 1

    @pl.when(i == 0)
    def _():
        acc_ref[0] = 0.0

    @pl.when(i < nlast)
    def _():
        acc_ref[0] = jnp.maximum(acc_ref[0], jnp.max(jnp.abs(w_ref[...])))

    @pl.when(i == nlast)
    def _():
        col = i * _SCALE_COLS + lax.broadcasted_iota(
            jnp.int32, (_EMB_DIM, _SCALE_COLS), 1
        )
        a = jnp.where(col < _NUM_EMB, jnp.abs(w_ref[...]), 0.0)
        acc_ref[0] = jnp.maximum(acc_ref[0], jnp.max(a))
        out_ref[0] = jnp.maximum(acc_ref[0], 1e-8) / 127.0


_scale_call = pl.pallas_call(
    _scale_body,
    grid=(pl.cdiv(_NUM_EMB, _SCALE_COLS),),
    in_specs=[pl.BlockSpec((_EMB_DIM, _SCALE_COLS), lambda i: (0, i))],
    out_specs=pl.BlockSpec(memory_space=pltpu.SMEM),
    out_shape=jax.ShapeDtypeStruct((1,), jnp.float32),
    scratch_shapes=[pltpu.SMEM((1,), jnp.float32)],
)

_QT_COLS = 1024
_K = _EMB_DIM * _NWIN


def _quant_body(scale_ref, *refs):
    w_refs = refs[:_NWIN]
    out_ref, sa_ref, sb_ref = refs[_NWIN], refs[_NWIN + 1], refs[_NWIN + 2]
    i = pl.program_id(0)

    @pl.when(i == 0)
    def _():
        r = lax.broadcasted_iota(jnp.int32, (_K, 128), 0)
        n = lax.broadcasted_iota(jnp.int32, (_K, 128), 1)
        same = (r >> 6) == (n >> 4)
        l, j4 = r & 63, (n & 15) * 4
        sa_ref[...] = jnp.where(same & (l == j4), 1.0, 0.0) + jnp.where(
            same & (l == j4 + 1), 256.0, 0.0
        )
        sb_ref[...] = jnp.where(same & (l == j4 + 2), 1.0, 0.0) + jnp.where(
            same & (l == j4 + 3), 256.0, 0.0
        )

    inv = 1.0 / scale_ref[0]
    q = jnp.concatenate([r[...] for r in w_refs], axis=0)
    q = jnp.clip(jnp.round(q * inv), -127.0, 126.0) + 128.0
    wa = lax.dot_general(q, sa_ref[...], (((0,), (0,)), ((), ())))
    wb = lax.dot_general(q, sb_ref[...], (((0,), (0,)), ((), ())))
    w = (wa.astype(jnp.int32) & 0xFFFF) | (wb.astype(jnp.int32) << 16)
    out_ref[...] = w ^ jnp.int32(-2139062144)


_quant_call = pl.pallas_call(
    _quant_body,
    grid=(_W // _QT_COLS,),
    in_specs=[pl.BlockSpec(memory_space=pltpu.SMEM)]
    + [
        pl.BlockSpec(
            (_EMB_DIM, _QT_COLS),
            functools.partial(
                lambda s, i: (0, jnp.minimum(s * (_W // _QT_COLS) + i, 976)), s
            ),
        )
        for s in range(_NWIN)
    ],
    out_specs=pl.BlockSpec((_QT_COLS, 128), lambda i: (i, 0)),
    out_shape=jax.ShapeDtypeStruct((_W, 128), jnp.int32),
    scratch_shapes=[
        pltpu.VMEM((_K, 128), jnp.float32),
        pltpu.VMEM((_K, 128), jnp.float32),
    ],
)

_NW = 32
_B_PER_W = _B // _NW
_CHUNK = 640
_NCHUNK = _B_PER_W // _CHUNK


_HIST = 50
_CHUNK_B = 16
_LK = _HIST * _CHUNK_B
_SUB = 400


def _gather_body(table_hbm, idx_hbm, out_hbm, idx_v, p_v, h_v, rows_v, out_v, sem):
    wid = lax.axis_index("s") * 2 + lax.axis_index("c")
    base = wid * _B_PER_W
    lanes = lax.iota(jnp.int32, 16)

    def chunk(c, carry):
        off = pl.multiple_of(base + c * _LK, 32)
        pltpu.sync_copy(idx_hbm.at[pl.ds(off, _LK)], idx_v)

        def prep(g, carry2):
            s = pl.multiple_of(g * 16, 16)
            iv = idx_v[pl.ds(s, 16)]
            p_v[pl.ds(s, 16)] = iv & (_W - 1)
            h_v[pl.ds(s, 16)] = lax.shift_right_logical(iv, 17) << 4
            return carry2

        lax.fori_loop(0, _LK // 16, prep, 0)

        for sub in range(_LK // _SUB):
            pltpu.async_copy(
                table_hbm.at[p_v.at[pl.ds(sub * _SUB, _SUB)]], rows_v, sem
            ).wait()

            def select(g, carry2):
                s = pl.multiple_of(g * 16, 16)
                r_vec = s + lanes
                k_vec = sub * _SUB + r_vec
                h16 = h_v[pl.ds(sub * _SUB + s, 16)]
                b_loc = lax.shift_right_logical(k_vec * 1311, 16)
                hh = k_vec - b_loc * _HIST
                o_base = hh << 4
                for j in range(16):
                    v = plsc.load_gather(rows_v, [r_vec, h16 + j])
                    plsc.store_scatter(out_v, [o_base + j, b_loc], v)
                return carry2

            lax.fori_loop(0, _SUB // 16, select, 0)

        col0 = pl.multiple_of(wid * 128 + c * _CHUNK_B, 16)
        pltpu.sync_copy(out_v, out_hbm.at[:, pl.ds(col0, _CHUNK_B)])
        return carry

    lax.fori_loop(0, _B_PER_W // _LK, chunk, 0)


_gather_call = functools.partial(
    pl.kernel,
    mesh=plsc.VectorSubcoreMesh(core_axis_name="c", subcore_axis_name="s"),
    compiler_params=pltpu.CompilerParams(
        use_tc_tiling_on_sc=False, needs_layout_passes=False
    ),
    out_type=jax.ShapeDtypeStruct((_HIST * 16, 4096), jnp.int32),
    scratch_types=[
        pltpu.VMEM((_LK,), jnp.int32),
        pltpu.VMEM((_LK,), jnp.int32),
        pltpu.VMEM((_LK,), jnp.int32),
        pltpu.VMEM((_SUB, 128), jnp.int32),
        pltpu.VMEM((_HIST * 16, _CHUNK_B), jnp.int32),
        pltpu.SemaphoreType.DMA,
    ],
)(_gather_body)


def kernel(x, weight):
    wt = weight.T
    scale = _scale_call(wt)
    packed = _quant_call(scale, *([wt] * _NWIN))
    rows = _gather_call(packed, x.reshape(-1))
    emb8 = lax.bitcast_convert_type(rows, jnp.int8)
    emb = (
        emb8.reshape(_HIST, 16, 4096, 4)
        .transpose(2, 0, 1, 3)
        .reshape(4096, _HIST, _EMB_DIM)
    )
    return emb, scale


# speedup vs baseline: 4.0739x; 1.0447x over previous
import functools

import jax
import jax.numpy as jnp
from jax import lax
from jax.experimental import pallas as pl
from jax.experimental.pallas import tpu as pltpu
from jax.experimental.pallas import tpu_sc as plsc

_NUM_EMB = 1000000
_EMB_DIM = 64
_B = 4096 * 50
_W = 131072
_NWIN = 8

_SCALE_COLS = 16384


def _scale_body(w_ref, out_ref, acc_ref):
    i = pl.program_id(0)
    nlast = pl.num_programs(0) - 1

    @pl.when(i == 0)
    def _():
        acc_ref[0] = 0.0

    @pl.when(i < nlast)
    def _():
        acc_ref[0] = jnp.maximum(acc_ref[0], jnp.max(jnp.abs(w_ref[...])))

    @pl.when(i == nlast)
    def _():
        col = i * _SCALE_COLS + lax.broadcasted_iota(
            jnp.int32, (_EMB_DIM, _SCALE_COLS), 1
        )
        a = jnp.where(col < _NUM_EMB, jnp.abs(w_ref[...]), 0.0)
        acc_ref[0] = jnp.maximum(acc_ref[0], jnp.max(a))
        out_ref[0] = jnp.maximum(acc_ref[0], 1e-8) / 127.0


_scale_call = pl.pallas_call(
    _scale_body,
    grid=(pl.cdiv(_NUM_EMB, _SCALE_COLS),),
    in_specs=[pl.BlockSpec((_EMB_DIM, _SCALE_COLS), lambda i: (0, i))],
    out_specs=pl.BlockSpec(memory_space=pltpu.SMEM),
    out_shape=jax.ShapeDtypeStruct((1,), jnp.float32),
    scratch_shapes=[pltpu.SMEM((1,), jnp.float32)],
)

_QT_COLS = 1024
_K = _EMB_DIM * _NWIN


def _quant_body(scale_ref, *refs):
    w_refs = refs[:_NWIN]
    out_ref, sa_ref, sb_ref = refs[_NWIN], refs[_NWIN + 1], refs[_NWIN + 2]
    i = pl.program_id(0)

    @pl.when(i == 0)
    def _():
        r = lax.broadcasted_iota(jnp.int32, (_K, 128), 0)
        n = lax.broadcasted_iota(jnp.int32, (_K, 128), 1)
        same = (r >> 6) == (n >> 4)
        l, j4 = r & 63, (n & 15) * 4
        sa_ref[...] = jnp.where(same & (l == j4), 1.0, 0.0) + jnp.where(
            same & (l == j4 + 1), 256.0, 0.0
        )
        sb_ref[...] = jnp.where(same & (l == j4 + 2), 1.0, 0.0) + jnp.where(
            same & (l == j4 + 3), 256.0, 0.0
        )

    inv = 1.0 / scale_ref[0]
    q = jnp.concatenate([r[...] for r in w_refs], axis=0)
    q = jnp.clip(jnp.round(q * inv), -127.0, 126.0) + 128.0
    wa = lax.dot_general(q, sa_ref[...], (((0,), (0,)), ((), ())))
    wb = lax.dot_general(q, sb_ref[...], (((0,), (0,)), ((), ())))
    w = (wa.astype(jnp.int32) & 0xFFFF) | (wb.astype(jnp.int32) << 16)
    out_ref[...] = w ^ jnp.int32(-2139062144)


_quant_call = pl.pallas_call(
    _quant_body,
    grid=(_W // _QT_COLS,),
    in_specs=[pl.BlockSpec(memory_space=pltpu.SMEM)]
    + [
        pl.BlockSpec(
            (_EMB_DIM, _QT_COLS),
            functools.partial(
                lambda s, i: (0, jnp.minimum(s * (_W // _QT_COLS) + i, 976)), s
            ),
        )
        for s in range(_NWIN)
    ],
    out_specs=pl.BlockSpec((_QT_COLS, 128), lambda i: (i, 0)),
    out_shape=jax.ShapeDtypeStruct((_W, 128), jnp.int32),
    scratch_shapes=[
        pltpu.VMEM((_K, 128), jnp.float32),
        pltpu.VMEM((_K, 128), jnp.float32),
    ],
)

_NW = 32
_B_PER_W = _B // _NW
_CHUNK = 640
_NCHUNK = _B_PER_W // _CHUNK


_HIST = 50
_CHUNK_B = 16
_LK = _HIST * _CHUNK_B
_SUB = 400


def _gather_body(
    table_hbm, idx_hbm, out_hbm, idx_v, p_v, h_v, rows_a, rows_b, out_v, sem
):
    wid = lax.axis_index("s") * 2 + lax.axis_index("c")
    base = wid * _B_PER_W
    lanes = lax.iota(jnp.int32, 16)
    rows_bufs = (rows_a, rows_b)
    nsub = _LK // _SUB

    def chunk(c, carry):
        off = pl.multiple_of(base + c * _LK, 32)
        pltpu.sync_copy(idx_hbm.at[pl.ds(off, _LK)], idx_v)

        def prep(g, carry2):
            s = pl.multiple_of(g * 16, 16)
            iv = idx_v[pl.ds(s, 16)]
            p_v[pl.ds(s, 16)] = iv & (_W - 1)
            h_v[pl.ds(s, 16)] = lax.shift_right_logical(iv, 17) << 4
            return carry2

        lax.fori_loop(0, _LK // 16, prep, 0)

        cps = [
            pltpu.make_async_copy(
                table_hbm.at[p_v.at[pl.ds(sub * _SUB, _SUB)]],
                rows_bufs[sub % 2],
                sem,
            )
            for sub in range(nsub)
        ]
        cps[0].start()
        for sub in range(nsub):
            cps[sub].wait()
            if sub + 1 < nsub:
                cps[sub + 1].start()
            rows_v = rows_bufs[sub % 2]

            def select(g, carry2):
                s = pl.multiple_of(g * 16, 16)
                r_vec = s + lanes
                k_vec = sub * _SUB + r_vec
                h16 = h_v[pl.ds(sub * _SUB + s, 16)]
                b_loc = lax.shift_right_logical(k_vec * 1311, 16)
                hh = k_vec - b_loc * _HIST
                o_base = hh << 4
                for j in range(16):
                    v = plsc.load_gather(rows_v, [r_vec, h16 + j])
                    plsc.store_scatter(out_v, [o_base + j, b_loc], v)
                return carry2

            lax.fori_loop(0, _SUB // 16, select, 0)

        col0 = pl.multiple_of(wid * 128 + c * _CHUNK_B, 16)
        pltpu.sync_copy(out_v, out_hbm.at[:, pl.ds(col0, _CHUNK_B)])
        return carry

    lax.fori_loop(0, _B_PER_W // _LK, chunk, 0)


_gather_call = functools.partial(
    pl.kernel,
    mesh=plsc.VectorSubcoreMesh(core_axis_name="c", subcore_axis_name="s"),
    compiler_params=pltpu.CompilerParams(
        use_tc_tiling_on_sc=False, needs_layout_passes=False
    ),
    out_type=jax.ShapeDtypeStruct((_HIST * 16, 4096), jnp.int32),
    scratch_types=[
        pltpu.VMEM((_LK,), jnp.int32),
        pltpu.VMEM((_LK,), jnp.int32),
        pltpu.VMEM((_LK,), jnp.int32),
        pltpu.VMEM((_SUB, 128), jnp.int32),
        pltpu.VMEM((_SUB, 128), jnp.int32),
        pltpu.VMEM((_HIST * 16, _CHUNK_B), jnp.int32),
        pltpu.SemaphoreType.DMA,
    ],
)(_gather_body)


def kernel(x, weight):
    wt = weight.T
    scale = _scale_call(wt)
    packed = _quant_call(scale, *([wt] * _NWIN))
    rows = _gather_call(packed, x.reshape(-1))
    emb8 = lax.bitcast_convert_type(rows, jnp.int8)
    emb = (
        emb8.reshape(_HIST, 16, 4096, 4)
        .transpose(2, 0, 1, 3)
        .reshape(4096, _HIST, _EMB_DIM)
    )
    return emb, scale


# speedup vs baseline: 4.3923x; 1.0782x over previous
import functools

import jax
import jax.numpy as jnp
from jax import lax
from jax.experimental import pallas as pl
from jax.experimental.pallas import tpu as pltpu
from jax.experimental.pallas import tpu_sc as plsc

_NUM_EMB = 1000000
_EMB_DIM = 64
_B = 4096 * 50
_W = 131072
_NWIN = 8

_SCALE_COLS = 16384


def _scale_body(w_ref, out_ref, acc_ref):
    i = pl.program_id(0)
    nlast = pl.num_programs(0) - 1

    @pl.when(i == 0)
    def _():
        acc_ref[0] = 0.0

    @pl.when(i < nlast)
    def _():
        acc_ref[0] = jnp.maximum(acc_ref[0], jnp.max(jnp.abs(w_ref[...])))

    @pl.when(i == nlast)
    def _():
        col = i * _SCALE_COLS + lax.broadcasted_iota(
            jnp.int32, (_EMB_DIM, _SCALE_COLS), 1
        )
        a = jnp.where(col < _NUM_EMB, jnp.abs(w_ref[...]), 0.0)
        acc_ref[0] = jnp.maximum(acc_ref[0], jnp.max(a))
        out_ref[0] = jnp.maximum(acc_ref[0], 1e-8) / 127.0


_scale_call = pl.pallas_call(
    _scale_body,
    grid=(pl.cdiv(_NUM_EMB, _SCALE_COLS),),
    in_specs=[pl.BlockSpec((_EMB_DIM, _SCALE_COLS), lambda i: (0, i))],
    out_specs=pl.BlockSpec(memory_space=pltpu.SMEM),
    out_shape=jax.ShapeDtypeStruct((1,), jnp.float32),
    scratch_shapes=[pltpu.SMEM((1,), jnp.float32)],
)

_QT_COLS = 2048
_K = _EMB_DIM * _NWIN


def _quant_body(scale_ref, *refs):
    w_refs = refs[:_NWIN]
    out_ref, sa_ref, sb_ref = refs[_NWIN], refs[_NWIN + 1], refs[_NWIN + 2]
    i = pl.program_id(0)

    @pl.when(i == 0)
    def _():
        r = lax.broadcasted_iota(jnp.int32, (_K, 128), 0)
        n = lax.broadcasted_iota(jnp.int32, (_K, 128), 1)
        same = (r >> 6) == (n >> 4)
        l, j4 = r & 63, (n & 15) * 4
        sa_ref[...] = jnp.where(same & (l == j4), 1.0, 0.0) + jnp.where(
            same & (l == j4 + 1), 256.0, 0.0
        )
        sb_ref[...] = jnp.where(same & (l == j4 + 2), 1.0, 0.0) + jnp.where(
            same & (l == j4 + 3), 256.0, 0.0
        )

    inv = 1.0 / scale_ref[0]
    q = jnp.concatenate([r[...] for r in w_refs], axis=0)
    q = jnp.clip(jnp.round(q * inv), -127.0, 126.0) + 128.0
    wa = lax.dot_general(q, sa_ref[...], (((0,), (0,)), ((), ())))
    wb = lax.dot_general(q, sb_ref[...], (((0,), (0,)), ((), ())))
    w = (wa.astype(jnp.int32) & 0xFFFF) | (wb.astype(jnp.int32) << 16)
    out_ref[...] = w ^ jnp.int32(-2139062144)


_quant_call = pl.pallas_call(
    _quant_body,
    grid=(_W // _QT_COLS,),
    in_specs=[pl.BlockSpec(memory_space=pltpu.SMEM)]
    + [
        pl.BlockSpec(
            (_EMB_DIM, _QT_COLS),
            functools.partial(
                lambda s, i: (
                    0,
                    jnp.minimum(s * (_W // _QT_COLS) + i, _NUM_EMB // _QT_COLS),
                ),
                s,
            ),
        )
        for s in range(_NWIN)
    ],
    out_specs=pl.BlockSpec((_QT_COLS, 128), lambda i: (i, 0)),
    out_shape=jax.ShapeDtypeStruct((_W, 128), jnp.int32),
    scratch_shapes=[
        pltpu.VMEM((_K, 128), jnp.float32),
        pltpu.VMEM((_K, 128), jnp.float32),
    ],
)

_NW = 32
_B_PER_W = _B // _NW
_CHUNK = 640
_NCHUNK = _B_PER_W // _CHUNK


_HIST = 50
_CHUNK_B = 16
_LK = _HIST * _CHUNK_B
_SUB = 400


def _gather_body(
    table_hbm, idx_hbm, out_hbm, idx_v, p_v, h_v, rows_a, rows_b, out_v, sem
):
    wid = lax.axis_index("s") * 2 + lax.axis_index("c")
    base = wid * _B_PER_W
    lanes = lax.iota(jnp.int32, 16)
    rows_bufs = (rows_a, rows_b)
    nsub = _LK // _SUB

    def chunk(c, carry):
        off = pl.multiple_of(base + c * _LK, 32)
        pltpu.sync_copy(idx_hbm.at[pl.ds(off, _LK)], idx_v)

        def prep(g, carry2):
            s = pl.multiple_of(g * 16, 16)
            iv = idx_v[pl.ds(s, 16)]
            p_v[pl.ds(s, 16)] = iv & (_W - 1)
            h_v[pl.ds(s, 16)] = lax.shift_right_logical(iv, 17) << 4
            return carry2

        lax.fori_loop(0, _LK // 16, prep, 0)

        cps = [
            pltpu.make_async_copy(
                table_hbm.at[p_v.at[pl.ds(sub * _SUB, _SUB)]],
                rows_bufs[sub % 2],
                sem,
            )
            for sub in range(nsub)
        ]
        cps[0].start()
        for sub in range(nsub):
            cps[sub].wait()
            if sub + 1 < nsub:
                cps[sub + 1].start()
            rows_v = rows_bufs[sub % 2]

            def select(g, carry2):
                s = pl.multiple_of(g * 16, 16)
                r_vec = s + lanes
                k_vec = sub * _SUB + r_vec
                h16 = h_v[pl.ds(sub * _SUB + s, 16)]
                b_loc = lax.shift_right_logical(k_vec * 1311, 16)
                hh = k_vec - b_loc * _HIST
                o_base = hh << 4
                for j in range(16):
                    v = plsc.load_gather(rows_v, [r_vec, h16 + j])
                    plsc.store_scatter(out_v, [o_base + j, b_loc], v)
                return carry2

            lax.fori_loop(0, _SUB // 16, select, 0)

        col0 = pl.multiple_of(wid * 128 + c * _CHUNK_B, 16)
        pltpu.sync_copy(out_v, out_hbm.at[:, pl.ds(col0, _CHUNK_B)])
        return carry

    lax.fori_loop(0, _B_PER_W // _LK, chunk, 0)


_gather_call = functools.partial(
    pl.kernel,
    mesh=plsc.VectorSubcoreMesh(core_axis_name="c", subcore_axis_name="s"),
    compiler_params=pltpu.CompilerParams(
        use_tc_tiling_on_sc=False, needs_layout_passes=False
    ),
    out_type=jax.ShapeDtypeStruct((_HIST * 16, 4096), jnp.int32),
    scratch_types=[
        pltpu.VMEM((_LK,), jnp.int32),
        pltpu.VMEM((_LK,), jnp.int32),
        pltpu.VMEM((_LK,), jnp.int32),
        pltpu.VMEM((_SUB, 128), jnp.int32),
        pltpu.VMEM((_SUB, 128), jnp.int32),
        pltpu.VMEM((_HIST * 16, _CHUNK_B), jnp.int32),
        pltpu.SemaphoreType.DMA,
    ],
)(_gather_body)


def kernel(x, weight):
    wt = weight.T
    scale = _scale_call(wt)
    packed = _quant_call(scale, *([wt] * _NWIN))
    rows = _gather_call(packed, x.reshape(-1))
    emb8 = lax.bitcast_convert_type(rows, jnp.int8)
    emb = (
        emb8.reshape(_HIST, 16, 4096, 4)
        .transpose(2, 0, 1, 3)
        .reshape(4096, _HIST, _EMB_DIM)
    )
    return emb, scale


# speedup vs baseline: 4.6719x; 1.0637x over previous
import functools

import jax
import jax.numpy as jnp
from jax import lax
from jax.experimental import pallas as pl
from jax.experimental.pallas import tpu as pltpu
from jax.experimental.pallas import tpu_sc as plsc

_NUM_EMB = 1000000
_EMB_DIM = 64
_B = 4096 * 50
_W = 131072
_NWIN = 8

_SCALE_COLS = 32768


def _scale_body(w_ref, out_ref, acc_ref):
    i = pl.program_id(0)
    nlast = pl.num_programs(0) - 1

    @pl.when(i == 0)
    def _():
        acc_ref[0] = 0.0

    @pl.when(i < nlast)
    def _():
        acc_ref[0] = jnp.maximum(acc_ref[0], jnp.max(jnp.abs(w_ref[...])))

    @pl.when(i == nlast)
    def _():
        col = i * _SCALE_COLS + lax.broadcasted_iota(
            jnp.int32, (_EMB_DIM, _SCALE_COLS), 1
        )
        a = jnp.where(col < _NUM_EMB, jnp.abs(w_ref[...]), 0.0)
        acc_ref[0] = jnp.maximum(acc_ref[0], jnp.max(a))
        out_ref[0] = jnp.maximum(acc_ref[0], 1e-8) / 127.0


_scale_call = pl.pallas_call(
    _scale_body,
    grid=(pl.cdiv(_NUM_EMB, _SCALE_COLS),),
    in_specs=[pl.BlockSpec((_EMB_DIM, _SCALE_COLS), lambda i: (0, i))],
    out_specs=pl.BlockSpec(memory_space=pltpu.SMEM),
    out_shape=jax.ShapeDtypeStruct((1,), jnp.float32),
    scratch_shapes=[pltpu.SMEM((1,), jnp.float32)],
)

_QT_COLS = 4096
_K = _EMB_DIM * _NWIN


def _quant_body(scale_ref, *refs):
    w_refs = refs[:_NWIN]
    out_ref, sa_ref, sb_ref = refs[_NWIN], refs[_NWIN + 1], refs[_NWIN + 2]
    i = pl.program_id(0)

    @pl.when(i == 0)
    def _():
        r = lax.broadcasted_iota(jnp.int32, (_K, 128), 0)
        n = lax.broadcasted_iota(jnp.int32, (_K, 128), 1)
        same = (r >> 6) == (n >> 4)
        l, j4 = r & 63, (n & 15) * 4
        sa_ref[...] = jnp.where(same & (l == j4), 1.0, 0.0) + jnp.where(
            same & (l == j4 + 1), 256.0, 0.0
        )
        sb_ref[...] = jnp.where(same & (l == j4 + 2), 1.0, 0.0) + jnp.where(
            same & (l == j4 + 3), 256.0, 0.0
        )

    inv = 1.0 / scale_ref[0]
    q = jnp.concatenate([r[...] for r in w_refs], axis=0)
    q = jnp.clip(jnp.round(q * inv), -127.0, 126.0) + 128.0
    wa = lax.dot_general(q, sa_ref[...], (((0,), (0,)), ((), ())))
    wb = lax.dot_general(q, sb_ref[...], (((0,), (0,)), ((), ())))
    w = (wa.astype(jnp.int32) & 0xFFFF) | (wb.astype(jnp.int32) << 16)
    out_ref[...] = w ^ jnp.int32(-2139062144)


_quant_call = pl.pallas_call(
    _quant_body,
    grid=(_W // _QT_COLS,),
    in_specs=[pl.BlockSpec(memory_space=pltpu.SMEM)]
    + [
        pl.BlockSpec(
            (_EMB_DIM, _QT_COLS),
            functools.partial(
                lambda s, i: (
                    0,
                    jnp.minimum(s * (_W // _QT_COLS) + i, _NUM_EMB // _QT_COLS),
                ),
                s,
            ),
        )
        for s in range(_NWIN)
    ],
    out_specs=pl.BlockSpec((_QT_COLS, 128), lambda i: (i, 0)),
    out_shape=jax.ShapeDtypeStruct((_W, 128), jnp.int32),
    scratch_shapes=[
        pltpu.VMEM((_K, 128), jnp.float32),
        pltpu.VMEM((_K, 128), jnp.float32),
    ],
)

_NW = 32
_B_PER_W = _B // _NW
_CHUNK = 640
_NCHUNK = _B_PER_W // _CHUNK


_HIST = 50
_CHUNK_B = 16
_LK = _HIST * _CHUNK_B
_SUB = 400


def _gather_body(
    table_hbm, idx_hbm, out_hbm, idx_v, p_v, h_v, rows_a, rows_b, out_v, sem
):
    wid = lax.axis_index("s") * 2 + lax.axis_index("c")
    base = wid * _B_PER_W
    lanes = lax.iota(jnp.int32, 16)
    rows_bufs = (rows_a, rows_b)
    nsub = _LK // _SUB

    def chunk(c, carry):
        off = pl.multiple_of(base + c * _LK, 32)
        pltpu.sync_copy(idx_hbm.at[pl.ds(off, _LK)], idx_v)

        def prep(g, carry2):
            s = pl.multiple_of(g * 16, 16)
            iv = idx_v[pl.ds(s, 16)]
            p_v[pl.ds(s, 16)] = iv & (_W - 1)
            h_v[pl.ds(s, 16)] = lax.shift_right_logical(iv, 17) << 4
            return carry2

        lax.fori_loop(0, _LK // 16, prep, 0)

        cps = [
            pltpu.make_async_copy(
                table_hbm.at[p_v.at[pl.ds(sub * _SUB, _SUB)]],
                rows_bufs[sub % 2],
                sem,
            )
            for sub in range(nsub)
        ]
        cps[0].start()
        for sub in range(nsub):
            cps[sub].wait()
            if sub + 1 < nsub:
                cps[sub + 1].start()
            rows_v = rows_bufs[sub % 2]

            def select(g, carry2):
                s = pl.multiple_of(g * 16, 16)
                r_vec = s + lanes
                k_vec = sub * _SUB + r_vec
                h16 = h_v[pl.ds(sub * _SUB + s, 16)]
                b_loc = lax.shift_right_logical(k_vec * 1311, 16)
                hh = k_vec - b_loc * _HIST
                o_base = hh << 4
                for j in range(16):
                    v = plsc.load_gather(rows_v, [r_vec, h16 + j])
                    plsc.store_scatter(out_v, [o_base + j, b_loc], v)
                return carry2

            lax.fori_loop(0, _SUB // 16, select, 0)

        col0 = pl.multiple_of(wid * 128 + c * _CHUNK_B, 16)
        pltpu.sync_copy(out_v, out_hbm.at[:, pl.ds(col0, _CHUNK_B)])
        return carry

    lax.fori_loop(0, _B_PER_W // _LK, chunk, 0)


_gather_call = functools.partial(
    pl.kernel,
    mesh=plsc.VectorSubcoreMesh(core_axis_name="c", subcore_axis_name="s"),
    compiler_params=pltpu.CompilerParams(
        use_tc_tiling_on_sc=False, needs_layout_passes=False
    ),
    out_type=jax.ShapeDtypeStruct((_HIST * 16, 4096), jnp.int32),
    scratch_types=[
        pltpu.VMEM((_LK,), jnp.int32),
        pltpu.VMEM((_LK,), jnp.int32),
        pltpu.VMEM((_LK,), jnp.int32),
        pltpu.VMEM((_SUB, 128), jnp.int32),
        pltpu.VMEM((_SUB, 128), jnp.int32),
        pltpu.VMEM((_HIST * 16, _CHUNK_B), jnp.int32),
        pltpu.SemaphoreType.DMA,
    ],
)(_gather_body)


def kernel(x, weight):
    wt = weight.T
    scale = _scale_call(wt)
    packed = _quant_call(scale, *([wt] * _NWIN))
    rows = _gather_call(packed, x.reshape(-1))
    emb8 = lax.bitcast_convert_type(rows, jnp.int8)
    emb = (
        emb8.reshape(_HIST, 16, 4096, 4)
        .transpose(2, 0, 1, 3)
        .reshape(4096, _HIST, _EMB_DIM)
    )
    return emb, scale


# speedup vs baseline: 4.7329x; 1.0131x over previous
import functools

import jax
import jax.numpy as jnp
from jax import lax
from jax.experimental import pallas as pl
from jax.experimental.pallas import tpu as pltpu
from jax.experimental.pallas import tpu_sc as plsc

_NUM_EMB = 1000000
_EMB_DIM = 64
_B = 4096 * 50
_W = 131072
_NWIN = 8

_SCALE_COLS = 65536


def _scale_body(w_ref, out_ref, acc_ref):
    i = pl.program_id(0)
    nlast = pl.num_programs(0) - 1

    @pl.when(i == 0)
    def _():
        acc_ref[0] = 0.0

    @pl.when(i < nlast)
    def _():
        acc_ref[0] = jnp.maximum(acc_ref[0], jnp.max(jnp.abs(w_ref[...])))

    @pl.when(i == nlast)
    def _():
        col = i * _SCALE_COLS + lax.broadcasted_iota(
            jnp.int32, (_EMB_DIM, _SCALE_COLS), 1
        )
        a = jnp.where(col < _NUM_EMB, jnp.abs(w_ref[...]), 0.0)
        acc_ref[0] = jnp.maximum(acc_ref[0], jnp.max(a))
        out_ref[0] = jnp.maximum(acc_ref[0], 1e-8) / 127.0


_scale_call = pl.pallas_call(
    _scale_body,
    grid=(pl.cdiv(_NUM_EMB, _SCALE_COLS),),
    in_specs=[pl.BlockSpec((_EMB_DIM, _SCALE_COLS), lambda i: (0, i))],
    out_specs=pl.BlockSpec(memory_space=pltpu.SMEM),
    out_shape=jax.ShapeDtypeStruct((1,), jnp.float32),
    scratch_shapes=[pltpu.SMEM((1,), jnp.float32)],
)

_QT_COLS = 8192
_K = _EMB_DIM * _NWIN


def _quant_body(scale_ref, *refs):
    w_refs = refs[:_NWIN]
    out_ref, sa_ref, sb_ref = refs[_NWIN], refs[_NWIN + 1], refs[_NWIN + 2]
    i = pl.program_id(0)

    @pl.when(i == 0)
    def _():
        r = lax.broadcasted_iota(jnp.int32, (_K, 128), 0)
        n = lax.broadcasted_iota(jnp.int32, (_K, 128), 1)
        same = (r >> 6) == (n >> 4)
        l, j4 = r & 63, (n & 15) * 4
        sa_ref[...] = jnp.where(same & (l == j4), 1.0, 0.0) + jnp.where(
            same & (l == j4 + 1), 256.0, 0.0
        )
        sb_ref[...] = jnp.where(same & (l == j4 + 2), 1.0, 0.0) + jnp.where(
            same & (l == j4 + 3), 256.0, 0.0
        )

    inv = 1.0 / scale_ref[0]
    q = jnp.concatenate([r[...] for r in w_refs], axis=0)
    q = jnp.clip(jnp.round(q * inv), -127.0, 126.0) + 128.0
    wa = lax.dot_general(q, sa_ref[...], (((0,), (0,)), ((), ())))
    wb = lax.dot_general(q, sb_ref[...], (((0,), (0,)), ((), ())))
    w = (wa.astype(jnp.int32) & 0xFFFF) | (wb.astype(jnp.int32) << 16)
    out_ref[...] = w ^ jnp.int32(-2139062144)


_quant_call = pl.pallas_call(
    _quant_body,
    grid=(_W // _QT_COLS,),
    in_specs=[pl.BlockSpec(memory_space=pltpu.SMEM)]
    + [
        pl.BlockSpec(
            (_EMB_DIM, _QT_COLS),
            functools.partial(
                lambda s, i: (
                    0,
                    jnp.minimum(s * (_W // _QT_COLS) + i, _NUM_EMB // _QT_COLS),
                ),
                s,
            ),
        )
        for s in range(_NWIN)
    ],
    out_specs=pl.BlockSpec((_QT_COLS, 128), lambda i: (i, 0)),
    out_shape=jax.ShapeDtypeStruct((_W, 128), jnp.int32),
    scratch_shapes=[
        pltpu.VMEM((_K, 128), jnp.float32),
        pltpu.VMEM((_K, 128), jnp.float32),
    ],
)

_NW = 32
_B_PER_W = _B // _NW
_CHUNK = 640
_NCHUNK = _B_PER_W // _CHUNK


_HIST = 50
_CHUNK_B = 16
_LK = _HIST * _CHUNK_B
_SUB = 400


def _gather_body(
    table_hbm, idx_hbm, out_hbm, idx_v, p_v, h_v, rows_a, rows_b, out_v, sem
):
    wid = lax.axis_index("s") * 2 + lax.axis_index("c")
    base = wid * _B_PER_W
    lanes = lax.iota(jnp.int32, 16)
    rows_bufs = (rows_a, rows_b)
    nsub = _LK // _SUB

    def chunk(c, carry):
        off = pl.multiple_of(base + c * _LK, 32)
        pltpu.sync_copy(idx_hbm.at[pl.ds(off, _LK)], idx_v)

        def prep(g, carry2):
            s = pl.multiple_of(g * 16, 16)
            iv = idx_v[pl.ds(s, 16)]
            p_v[pl.ds(s, 16)] = iv & (_W - 1)
            h_v[pl.ds(s, 16)] = lax.shift_right_logical(iv, 17) << 4
            return carry2

        lax.fori_loop(0, _LK // 16, prep, 0)

        cps = [
            pltpu.make_async_copy(
                table_hbm.at[p_v.at[pl.ds(sub * _SUB, _SUB)]],
                rows_bufs[sub % 2],
                sem,
            )
            for sub in range(nsub)
        ]
        cps[0].start()
        for sub in range(nsub):
            cps[sub].wait()
            if sub + 1 < nsub:
                cps[sub + 1].start()
            rows_v = rows_bufs[sub % 2]

            def select(g, carry2):
                s = pl.multiple_of(g * 16, 16)
                r_vec = s + lanes
                k_vec = sub * _SUB + r_vec
                h16 = h_v[pl.ds(sub * _SUB + s, 16)]
                b_loc = lax.shift_right_logical(k_vec * 1311, 16)
                hh = k_vec - b_loc * _HIST
                o_base = hh << 4
                for j in range(16):
                    v = plsc.load_gather(rows_v, [r_vec, h16 + j])
                    plsc.store_scatter(out_v, [o_base + j, b_loc], v)
                return carry2

            lax.fori_loop(0, _SUB // 16, select, 0)

        col0 = pl.multiple_of(wid * 128 + c * _CHUNK_B, 16)
        pltpu.sync_copy(out_v, out_hbm.at[:, pl.ds(col0, _CHUNK_B)])
        return carry

    lax.fori_loop(0, _B_PER_W // _LK, chunk, 0)


_gather_call = functools.partial(
    pl.kernel,
    mesh=plsc.VectorSubcoreMesh(core_axis_name="c", subcore_axis_name="s"),
    compiler_params=pltpu.CompilerParams(
        use_tc_tiling_on_sc=False, needs_layout_passes=False
    ),
    out_type=jax.ShapeDtypeStruct((_HIST * 16, 4096), jnp.int32),
    scratch_types=[
        pltpu.VMEM((_LK,), jnp.int32),
        pltpu.VMEM((_LK,), jnp.int32),
        pltpu.VMEM((_LK,), jnp.int32),
        pltpu.VMEM((_SUB, 128), jnp.int32),
        pltpu.VMEM((_SUB, 128), jnp.int32),
        pltpu.VMEM((_HIST * 16, _CHUNK_B), jnp.int32),
        pltpu.SemaphoreType.DMA,
    ],
)(_gather_body)


def kernel(x, weight):
    wt = weight.T
    scale = _scale_call(wt)
    packed = _quant_call(scale, *([wt] * _NWIN))
    rows = _gather_call(packed, x.reshape(-1))
    emb8 = lax.bitcast_convert_type(rows, jnp.int8)
    emb = (
        emb8.reshape(_HIST, 16, 4096, 4)
        .transpose(2, 0, 1, 3)
        .reshape(4096, _HIST, _EMB_DIM)
    )
    return emb, scale
